# baseline (speedup 1.0000x reference)
import functools

import jax
import jax.numpy as jnp
from jax.experimental import pallas as pl
from jax.experimental.pallas import tpu as pltpu

_HEADS = 6
_TOPK = 5


def _pool_body(rowa_ref, e_ref, m_ref, o_ref):
    e = e_ref[0]
    m = m_ref[0]
    o_ref[0] = jax.lax.dot_general(
        m, e, (((1,), (0,)), ((), ())),
        preferred_element_type=jnp.float32)


def _pooling(input_embed, batch_aa_mask, map_AA):
    n_aa, t_dim = batch_aa_mask.shape
    d = input_embed.shape[-1]
    mask3 = batch_aa_mask.reshape(n_aa, 1, t_dim)
    spec = pltpu.PrefetchScalarGridSpec(
        num_scalar_prefetch=1,
        grid=(n_aa,),
        in_specs=[
            pl.BlockSpec((1, t_dim, d), lambda i, rowa: (rowa[i], 0, 0)),
            pl.BlockSpec((1, 1, t_dim), lambda i, rowa: (i, 0, 0)),
        ],
        out_specs=pl.BlockSpec((1, 1, d), lambda i, rowa: (i, 0, 0)),
    )
    out = pl.pallas_call(
        _pool_body,
        grid_spec=spec,
        out_shape=jax.ShapeDtypeStruct((n_aa, 1, d), jnp.float32),
    )(map_AA, input_embed, mask3)
    return out.reshape(n_aa, d)


def _dense_body(l_max, d, *refs):
    gathered = refs[1:1 + l_max]
    m_ref = refs[1 + l_max]
    wq_ref, bq_ref, wk_ref, bk_ref, wg_ref, bg_ref = refs[2 + l_max:8 + l_max]
    o_ref = refs[8 + l_max]

    x = jnp.concatenate([r[0] for r in gathered], axis=0)
    mask = m_ref[0]

    def proj(w_ref, b_ref):
        return jax.lax.dot_general(
            x, w_ref[...], (((1,), (1,)), ((), ())),
            preferred_element_type=jnp.float32) + b_ref[...]

    q = proj(wq_ref, bq_ref)
    k = proj(wk_ref, bk_ref)
    dk = d // _HEADS
    scale = 1.0 / (dk ** 0.5)
    neg = jnp.float32(-1e9)

    attn_sum = jnp.zeros((l_max, l_max), jnp.float32)
    for h in range(_HEADS):
        qh = q[:, h * dk:(h + 1) * dk]
        kh = k[:, h * dk:(h + 1) * dk]
        s = jax.lax.dot_general(
            qh, kh, (((1,), (1,)), ((), ())),
            preferred_element_type=jnp.float32) * scale
        s = jnp.where(mask == 0.0, neg, s)
        s = s - jnp.max(s, axis=1, keepdims=True)
        e = jnp.exp(s)
        attn_sum = attn_sum + e / jnp.sum(e, axis=1, keepdims=True)

    mm = mask.reshape(l_max, 1) * mask
    adj = (attn_sum / _HEADS) * mm

    col = jax.lax.broadcasted_iota(jnp.int32, (l_max, l_max), 1)
    work = adj
    for _ in range(_TOPK - 1):
        mx = jnp.max(work, axis=1, keepdims=True)
        eq = work == mx
        first = jnp.min(jnp.where(eq, col, l_max), axis=1, keepdims=True)
        work = jnp.where(col == first, neg, work)
    kth = jnp.max(work, axis=1, keepdims=True)
    adj = jnp.where(adj >= kth, adj, 0.0)

    denom = jnp.sum(adj, axis=1, keepdims=True) + 1.0
    ax = jax.lax.dot_general(
        adj, x, (((1,), (0,)), ((), ())),
        preferred_element_type=jnp.float32)
    g = jax.lax.dot_general(
        ax, wg_ref[...], (((1,), (1,)), ((), ())),
        preferred_element_type=jnp.float32) + bg_ref[...]
    o_ref[0] = jnp.maximum(g / denom, 0.0)


def _dense(src, gidx, mask_indi, Wq, bq, Wk, bk, Wg, bg):
    b_max, l_max = mask_indi.shape
    d = src.shape[-1]
    mask3 = mask_indi.reshape(b_max, 1, l_max)

    def gspec(l):
        return pl.BlockSpec(
            (1, 1, d), lambda i, gidx, l=l: (gidx[i * l_max + l], 0, 0))

    wspec = pl.BlockSpec((d, d), lambda i, gidx: (0, 0))
    bspec = pl.BlockSpec((1, d), lambda i, gidx: (0, 0))
    spec = pltpu.PrefetchScalarGridSpec(
        num_scalar_prefetch=1,
        grid=(b_max,),
        in_specs=(
            [gspec(l) for l in range(l_max)]
            + [pl.BlockSpec((1, 1, l_max), lambda i, gidx: (i, 0, 0))]
            + [wspec, bspec, wspec, bspec, wspec, bspec]
        ),
        out_specs=pl.BlockSpec((1, l_max, d), lambda i, gidx: (i, 0, 0)),
    )
    out = pl.pallas_call(
        functools.partial(_dense_body, l_max, d),
        grid_spec=spec,
        out_shape=jax.ShapeDtypeStruct((b_max, l_max, d), jnp.float32),
    )(gidx, *([src] * l_max), mask3,
      Wq, bq.reshape(1, d), Wk, bk.reshape(1, d), Wg, bg.reshape(1, d))
    return out


def _route_body(g, *refs):
    need_ref = refs[1]
    gath = refs[2:2 + g]
    c_ref = refs[2 + g]
    o_ref = refs[3 + g]
    t = pl.program_id(0)
    for j in range(g):
        nd = (need_ref[t * g + j] > 0).astype(jnp.float32)
        o_ref[j] = nd * gath[j][0] + (1.0 - nd) * c_ref[j]


def _route(out_flat, clause3, gidx, need, g=4):
    n_as = clause3.shape[0]
    d = clause3.shape[-1]
    grid = n_as // g

    def gspec(j):
        return pl.BlockSpec(
            (1, 1, d), lambda i, gidx, need, j=j: (gidx[i * g + j], 0, 0))

    spec = pltpu.PrefetchScalarGridSpec(
        num_scalar_prefetch=2,
        grid=(grid,),
        in_specs=(
            [gspec(j) for j in range(g)]
            + [pl.BlockSpec((g, 1, d), lambda i, gidx, need: (i, 0, 0))]
        ),
        out_specs=pl.BlockSpec((g, 1, d), lambda i, gidx, need: (i, 0, 0)),
    )
    out = pl.pallas_call(
        functools.partial(_route_body, g),
        grid_spec=spec,
        out_shape=jax.ShapeDtypeStruct((n_as, 1, d), jnp.float32),
    )(gidx, need, *([out_flat] * g), clause3)
    return out.reshape(n_as, d)


def kernel(input_embed, clause_output, batch_aa_mask, aa_graph_length,
           map_AA, map_AA_idx, map_AS, map_AS_idx,
           Wq, bq, Wk, bk, Wg, bg):
    b_max, t_dim, d = input_embed.shape
    n_aa = map_AA.shape[0]
    n_as = map_AS.shape[0]
    l_max = 24

    map_AA = map_AA.astype(jnp.int32)
    map_AA_idx = map_AA_idx.astype(jnp.int32)
    map_AS = map_AS.astype(jnp.int32)
    map_AS_idx = map_AS_idx.astype(jnp.int32)
    aa_len = aa_graph_length.astype(jnp.int32)

    B = jnp.max(map_AS) + 1
    L = jnp.max(map_AA_idx) + 1

    row_aa = jnp.where(map_AA < B, map_AA, b_max)
    slot1 = row_aa * (l_max + 1) + map_AA_idx
    col_as = jnp.where(map_AS_idx < L, map_AS_idx, l_max)
    slot2 = map_AS * (l_max + 1) + col_as
    pri = jnp.concatenate([jnp.arange(n_aa, dtype=jnp.int32),
                           n_aa + jnp.arange(n_as, dtype=jnp.int32)])
    win = jnp.full(((b_max + 1) * (l_max + 1),), -1, jnp.int32)
    win = win.at[jnp.concatenate([slot1, slot2])].max(pri)
    src_idx = jnp.where(win >= 0, win, n_aa + n_as)
    gidx = src_idx.reshape(b_max + 1, l_max + 1)[:b_max, :l_max].reshape(-1)

    aa_embed = _pooling(input_embed, batch_aa_mask, map_AA)

    src = jnp.concatenate(
        [aa_embed, clause_output, jnp.zeros((1, d), jnp.float32)], axis=0)
    src = src.reshape(n_aa + n_as + 1, 1, d)

    pos = jnp.arange(l_max, dtype=jnp.int32)[None, :]
    mask_indi = (~((pos >= aa_len[:, None]) | (pos >= L))).astype(jnp.float32)

    out = _dense(src, gidx, mask_indi, Wq, bq, Wk, bk, Wg, bg)

    route_idx = map_AS * l_max + jnp.minimum(map_AS_idx, L - 1)
    need = (aa_len[map_AS] > 1).astype(jnp.int32)
    out_flat = out.reshape(b_max * l_max, 1, d)
    clause3 = clause_output.reshape(n_as, 1, d)
    return _route(out_flat, clause3, route_idx, need)


# speedup vs baseline: 6.6983x; 6.6983x over previous
import functools

import jax
import jax.numpy as jnp
from jax import lax
from jax.experimental import pallas as pl
from jax.experimental.pallas import tpu as pltpu
from jax.experimental.pallas import tpu_sc as plsc

_HEADS = 6
_TOPK = 5
_DP = 384
_G1 = 32
_BB = 64
_NC = 2
_NS = 16
_CHUNK = 128


def _pool_body(g, d, *refs):
    m_ref = refs[1 + g]
    o_ref = refs[2 + g]
    pad = jnp.zeros((1, _DP - d), jnp.float32)
    for j in range(g):
        e = refs[1 + j][0]
        m = m_ref[j]
        s = lax.dot_general(m, e, (((1,), (0,)), ((), ())),
                            preferred_element_type=jnp.float32)
        o_ref[j] = jnp.concatenate([s, pad], axis=1)


def _pooling(input_embed, batch_aa_mask, map_AA):
    n_aa, t_dim = batch_aa_mask.shape
    d = input_embed.shape[-1]
    mask3 = batch_aa_mask.reshape(n_aa, 1, t_dim)

    def espec(j):
        return pl.BlockSpec(
            (1, t_dim, d), lambda i, rowa, j=j: (rowa[i * _G1 + j], 0, 0))

    spec = pltpu.PrefetchScalarGridSpec(
        num_scalar_prefetch=1,
        grid=(n_aa // _G1,),
        in_specs=(
            [espec(j) for j in range(_G1)]
            + [pl.BlockSpec((_G1, 1, t_dim), lambda i, rowa: (i, 0, 0))]
        ),
        out_specs=pl.BlockSpec((_G1, 1, _DP), lambda i, rowa: (i, 0, 0)),
    )
    out = pl.pallas_call(
        functools.partial(_pool_body, _G1, d),
        grid_spec=spec,
        out_shape=jax.ShapeDtypeStruct((n_aa, 1, _DP), jnp.float32),
    )(map_AA, *([input_embed] * _G1), mask3)
    return out.reshape(n_aa, _DP)


def _sc_gather(table, idx):
    n = idx.shape[0]
    per_w = n // (_NC * _NS)
    n_chunks = per_w // _CHUNK
    assert per_w % _CHUNK == 0

    @functools.partial(
        pl.kernel,
        mesh=plsc.VectorSubcoreMesh(core_axis_name="c", subcore_axis_name="s"),
        out_type=jax.ShapeDtypeStruct((n, _DP), jnp.float32),
        scratch_types=[
            pltpu.VMEM((_CHUNK,), jnp.int32),
            pltpu.VMEM((_CHUNK, _DP), jnp.float32),
            pltpu.SemaphoreType.DMA,
        ],
    )
    def k(table_hbm, idx_hbm, out_hbm, idx_v, rows_v, sem):
        wid = lax.axis_index("s") * _NC + lax.axis_index("c")
        for c in range(n_chunks):
            base = wid * per_w + c * _CHUNK
            pltpu.sync_copy(idx_hbm.at[pl.ds(base, _CHUNK)], idx_v)
            pltpu.async_copy(table_hbm.at[idx_v], rows_v, sem).wait()
            pltpu.sync_copy(rows_v, out_hbm.at[pl.ds(base, _CHUNK)])

    return k(table, idx)


def _dense_body(l_max, d, x_ref, m_ref, wq_ref, bq_ref, wk_ref, bk_ref,
                wg_ref, bg_ref, o_ref):
    bb = _BB
    x = x_ref[...][:, :, :d]
    xf = x.reshape(bb * l_max, d)
    mask = m_ref[...].reshape(bb, 1, l_max)

    def proj(w_ref, b_ref):
        r = lax.dot_general(xf, w_ref[...], (((1,), (1,)), ((), ())),
                            preferred_element_type=jnp.float32) + b_ref[...]
        return r.reshape(bb, l_max, d)

    q = proj(wq_ref, bq_ref)
    k = proj(wk_ref, bk_ref)
    dk = d // _HEADS
    scale = 1.0 / (dk ** 0.5)
    neg = jnp.float32(-1e9)

    attn_sum = jnp.zeros((bb, l_max, l_max), jnp.float32)
    for h in range(_HEADS):
        qh = q[:, :, h * dk:(h + 1) * dk]
        kh = k[:, :, h * dk:(h + 1) * dk]
        s = lax.dot_general(
            qh, kh, (((2,), (2,)), ((0,), (0,))),
            preferred_element_type=jnp.float32) * scale
        s = jnp.where(mask == 0.0, neg, s)
        s = s - jnp.max(s, axis=2, keepdims=True)
        e = jnp.exp(s)
        attn_sum = attn_sum + e / jnp.sum(e, axis=2, keepdims=True)

    mm = mask.reshape(bb, l_max, 1) * mask
    adj = (attn_sum / _HEADS) * mm

    col = lax.broadcasted_iota(jnp.int32, (bb, l_max, l_max), 2)
    work = adj
    for _ in range(_TOPK - 1):
        mx = jnp.max(work, axis=2, keepdims=True)
        eq = work == mx
        first = jnp.min(jnp.where(eq, col, l_max), axis=2, keepdims=True)
        work = jnp.where(col == first, neg, work)
    kth = jnp.max(work, axis=2, keepdims=True)
    adj = jnp.where(adj >= kth, adj, 0.0)

    denom = jnp.sum(adj, axis=2, keepdims=True) + 1.0
    ax = lax.dot_general(
        adj, x, (((2,), (1,)), ((0,), (0,))),
        preferred_element_type=jnp.float32)
    g = lax.dot_general(
        ax.reshape(bb * l_max, d), wg_ref[...], (((1,), (1,)), ((), ())),
        preferred_element_type=jnp.float32) + bg_ref[...]
    g = jnp.maximum(g.reshape(bb, l_max, d) / denom, 0.0)
    o_ref[...] = jnp.concatenate(
        [g, jnp.zeros((bb, l_max, _DP - d), jnp.float32)], axis=2)


def _dense(inner, mask_indi, Wq, bq, Wk, bk, Wg, bg):
    b_max, l_max = mask_indi.shape
    d = Wq.shape[0]
    mask3 = mask_indi.reshape(b_max, 1, l_max)
    wspec = pl.BlockSpec((d, d), lambda i: (0, 0))
    bspec = pl.BlockSpec((1, d), lambda i: (0, 0))
    out = pl.pallas_call(
        functools.partial(_dense_body, l_max, d),
        grid=(b_max // _BB,),
        in_specs=[
            pl.BlockSpec((_BB, l_max, _DP), lambda i: (i, 0, 0)),
            pl.BlockSpec((_BB, 1, l_max), lambda i: (i, 0, 0)),
            wspec, bspec, wspec, bspec, wspec, bspec,
        ],
        out_specs=pl.BlockSpec((_BB, l_max, _DP), lambda i: (i, 0, 0)),
        out_shape=jax.ShapeDtypeStruct((b_max, l_max, _DP), jnp.float32),
    )(inner.reshape(b_max, l_max, _DP), mask3,
      Wq, bq.reshape(1, d), Wk, bk.reshape(1, d), Wg, bg.reshape(1, d))
    return out


def kernel(input_embed, clause_output, batch_aa_mask, aa_graph_length,
           map_AA, map_AA_idx, map_AS, map_AS_idx,
           Wq, bq, Wk, bk, Wg, bg):
    b_max, t_dim, d = input_embed.shape
    n_aa = map_AA.shape[0]
    n_as = map_AS.shape[0]
    l_max = 24

    map_AA = map_AA.astype(jnp.int32)
    map_AA_idx = map_AA_idx.astype(jnp.int32)
    map_AS = map_AS.astype(jnp.int32)
    map_AS_idx = map_AS_idx.astype(jnp.int32)
    aa_len = aa_graph_length.astype(jnp.int32)

    B = jnp.max(map_AS) + 1
    L = jnp.max(map_AA_idx) + 1

    row_aa = jnp.where(map_AA < B, map_AA, b_max)
    slot1 = row_aa * (l_max + 1) + map_AA_idx
    col_as = jnp.where(map_AS_idx < L, map_AS_idx, l_max)
    slot2 = map_AS * (l_max + 1) + col_as
    pri = jnp.concatenate([jnp.arange(n_aa, dtype=jnp.int32),
                           n_aa + jnp.arange(n_as, dtype=jnp.int32)])
    win = jnp.full(((b_max + 1) * (l_max + 1),), -1, jnp.int32)
    win = win.at[jnp.concatenate([slot1, slot2])].max(pri)
    src_idx = jnp.where(win >= 0, win, n_aa + n_as)
    gidx = src_idx.reshape(b_max + 1, l_max + 1)[:b_max, :l_max].reshape(-1)

    aa_embed = _pooling(input_embed, batch_aa_mask, map_AA)

    clause_pad = jnp.pad(clause_output, ((0, 0), (0, _DP - d)))
    src = jnp.concatenate(
        [aa_embed, clause_pad, jnp.zeros((1, _DP), jnp.float32)], axis=0)

    inner = _sc_gather(src, gidx)

    pos = jnp.arange(l_max, dtype=jnp.int32)[None, :]
    mask_indi = (~((pos >= aa_len[:, None]) | (pos >= L))).astype(jnp.float32)

    out = _dense(inner, mask_indi, Wq, bq, Wk, bk, Wg, bg)

    route_idx = map_AS * l_max + jnp.minimum(map_AS_idx, L - 1)
    need = aa_len[map_AS] > 1
    j = jnp.arange(n_as, dtype=jnp.int32)
    final_idx = jnp.where(need, route_idx, b_max * l_max + j)
    table = jnp.concatenate([out.reshape(b_max * l_max, _DP), clause_pad],
                            axis=0)
    res = _sc_gather(table, final_idx)
    return res[:, :d]


# speedup vs baseline: 11.8462x; 1.7685x over previous
import functools

import jax
import jax.numpy as jnp
from jax import lax
from jax.experimental import pallas as pl
from jax.experimental.pallas import tpu as pltpu
from jax.experimental.pallas import tpu_sc as plsc

_HEADS = 6
_TOPK = 5
_DP = 384
_G1 = 32
_BB = 64
_NC = 2
_NS = 16
_CHUNK = 128
_ZROWS = 2048


def _pool_body(g, d, *refs):
    m_ref = refs[1 + g]
    o_ref = refs[2 + g]
    pad = jnp.zeros((1, _DP - d), jnp.float32)
    for j in range(g):
        e = refs[1 + j][0]
        m = m_ref[j]
        s = lax.dot_general(m, e, (((1,), (0,)), ((), ())),
                            preferred_element_type=jnp.float32)
        o_ref[j] = jnp.concatenate([s, pad], axis=1)


def _pooling(input_embed, batch_aa_mask, map_AA):
    n_aa, t_dim = batch_aa_mask.shape
    d = input_embed.shape[-1]
    mask3 = batch_aa_mask.reshape(n_aa, 1, t_dim)

    def espec(j):
        return pl.BlockSpec(
            (1, t_dim, d), lambda i, rowa, j=j: (rowa[i * _G1 + j], 0, 0))

    spec = pltpu.PrefetchScalarGridSpec(
        num_scalar_prefetch=1,
        grid=(n_aa // _G1,),
        in_specs=(
            [espec(j) for j in range(_G1)]
            + [pl.BlockSpec((_G1, 1, t_dim), lambda i, rowa: (i, 0, 0))]
        ),
        out_specs=pl.BlockSpec((_G1, 1, _DP), lambda i, rowa: (i, 0, 0)),
    )
    out = pl.pallas_call(
        functools.partial(_pool_body, _G1, d),
        grid_spec=spec,
        out_shape=jax.ShapeDtypeStruct((n_aa, 1, _DP), jnp.float32),
    )(map_AA, *([input_embed] * _G1), mask3)
    return out.reshape(n_aa, _DP)


def _sc_gather(table, idx):
    n = idx.shape[0]
    per_w = n // (_NC * _NS)
    n_chunks = per_w // _CHUNK
    assert per_w % _CHUNK == 0

    @functools.partial(
        pl.kernel,
        mesh=plsc.VectorSubcoreMesh(core_axis_name="c", subcore_axis_name="s"),
        out_type=jax.ShapeDtypeStruct((n, _DP), jnp.float32),
        scratch_types=[
            pltpu.VMEM((_CHUNK,), jnp.int32),
            pltpu.VMEM((_CHUNK, _DP), jnp.float32),
            pltpu.SemaphoreType.DMA,
        ],
    )
    def k(table_hbm, idx_hbm, out_hbm, idx_v, rows_v, sem):
        wid = lax.axis_index("s") * _NC + lax.axis_index("c")
        for c in range(n_chunks):
            base = wid * per_w + c * _CHUNK
            pltpu.sync_copy(idx_hbm.at[pl.ds(base, _CHUNK)], idx_v)
            pltpu.async_copy(table_hbm.at[idx_v], rows_v, sem).wait()
            pltpu.sync_copy(rows_v, out_hbm.at[pl.ds(base, _CHUNK)])

    return k(table, idx)


def _dense_body(l_max, d, x_ref, m_ref, wq_ref, bq_ref, wk_ref, bk_ref,
                wg_ref, bg_ref, o_ref):
    bb = _BB
    x = x_ref[...][:, :, :d]
    xf = x.reshape(bb * l_max, d)
    mask = m_ref[...].reshape(bb, 1, l_max)

    def proj(w_ref, b_ref):
        r = lax.dot_general(xf, w_ref[...], (((1,), (1,)), ((), ())),
                            preferred_element_type=jnp.float32) + b_ref[...]
        return r.reshape(bb, l_max, d)

    q = proj(wq_ref, bq_ref)
    k = proj(wk_ref, bk_ref)
    dk = d // _HEADS
    scale = 1.0 / (dk ** 0.5)
    neg = jnp.float32(-1e9)

    attn_sum = jnp.zeros((bb, l_max, l_max), jnp.float32)
    for h in range(_HEADS):
        qh = q[:, :, h * dk:(h + 1) * dk]
        kh = k[:, :, h * dk:(h + 1) * dk]
        s = lax.dot_general(
            qh, kh, (((2,), (2,)), ((0,), (0,))),
            preferred_element_type=jnp.float32) * scale
        s = jnp.where(mask == 0.0, neg, s)
        s = s - jnp.max(s, axis=2, keepdims=True)
        e = jnp.exp(s)
        attn_sum = attn_sum + e / jnp.sum(e, axis=2, keepdims=True)

    mm = mask.reshape(bb, l_max, 1) * mask
    adj = (attn_sum / _HEADS) * mm

    col = lax.broadcasted_iota(jnp.int32, (bb, l_max, l_max), 2)
    work = adj
    for _ in range(_TOPK - 1):
        mx = jnp.max(work, axis=2, keepdims=True)
        eq = work == mx
        first = jnp.min(jnp.where(eq, col, l_max), axis=2, keepdims=True)
        work = jnp.where(col == first, neg, work)
    kth = jnp.max(work, axis=2, keepdims=True)
    adj = jnp.where(adj >= kth, adj, 0.0)

    denom = jnp.sum(adj, axis=2, keepdims=True) + 1.0
    ax = lax.dot_general(
        adj, x, (((2,), (1,)), ((0,), (0,))),
        preferred_element_type=jnp.float32)
    g = lax.dot_general(
        ax.reshape(bb * l_max, d), wg_ref[...], (((1,), (1,)), ((), ())),
        preferred_element_type=jnp.float32) + bg_ref[...]
    g = jnp.maximum(g.reshape(bb, l_max, d) / denom, 0.0)
    o_ref[...] = jnp.concatenate(
        [g, jnp.zeros((bb, l_max, _DP - d), jnp.float32)], axis=2)


def _dense(inner, mask_indi, Wq, bq, Wk, bk, Wg, bg):
    b_max, l_max = mask_indi.shape
    d = Wq.shape[0]
    mask3 = mask_indi.reshape(b_max, 1, l_max)
    wspec = pl.BlockSpec((d, d), lambda i: (0, 0))
    bspec = pl.BlockSpec((1, d), lambda i: (0, 0))
    out = pl.pallas_call(
        functools.partial(_dense_body, l_max, d),
        grid=(b_max // _BB,),
        in_specs=[
            pl.BlockSpec((_BB, l_max, _DP), lambda i: (i, 0, 0)),
            pl.BlockSpec((_BB, 1, l_max), lambda i: (i, 0, 0)),
            wspec, bspec, wspec, bspec, wspec, bspec,
        ],
        out_specs=pl.BlockSpec((_BB, l_max, _DP), lambda i: (i, 0, 0)),
        out_shape=jax.ShapeDtypeStruct((b_max, l_max, _DP), jnp.float32),
    )(inner.reshape(b_max, l_max, _DP), mask3,
      Wq, bq.reshape(1, d), Wk, bk.reshape(1, d), Wg, bg.reshape(1, d))
    return out


def kernel(input_embed, clause_output, batch_aa_mask, aa_graph_length,
           map_AA, map_AA_idx, map_AS, map_AS_idx,
           Wq, bq, Wk, bk, Wg, bg):
    b_max, t_dim, d = input_embed.shape
    n_aa = map_AA.shape[0]
    n_as = map_AS.shape[0]
    l_max = 24

    map_AA = map_AA.astype(jnp.int32)
    map_AA_idx = map_AA_idx.astype(jnp.int32)
    map_AS = map_AS.astype(jnp.int32)
    map_AS_idx = map_AS_idx.astype(jnp.int32)
    aa_len = aa_graph_length.astype(jnp.int32)

    B = jnp.max(map_AS) + 1
    L = jnp.max(map_AA_idx) + 1

    row_aa = jnp.where(map_AA < B, map_AA, b_max)
    slot1 = row_aa * (l_max + 1) + map_AA_idx
    col_as = jnp.where(map_AS_idx < L, map_AS_idx, l_max)
    slot2 = map_AS * (l_max + 1) + col_as
    pri = jnp.concatenate([jnp.arange(n_aa, dtype=jnp.int32),
                           n_aa + jnp.arange(n_as, dtype=jnp.int32)])
    win = jnp.full(((b_max + 1) * (l_max + 1),), -1, jnp.int32)
    win = win.at[jnp.concatenate([slot1, slot2])].max(pri)
    nslots = (b_max + 1) * (l_max + 1)
    zspread = n_aa + n_as + (jnp.arange(nslots, dtype=jnp.int32) % _ZROWS)
    src_idx = jnp.where(win >= 0, win, zspread)
    gidx = src_idx.reshape(b_max + 1, l_max + 1)[:b_max, :l_max].reshape(-1)

    aa_embed = _pooling(input_embed, batch_aa_mask, map_AA)

    clause_pad = jnp.pad(clause_output, ((0, 0), (0, _DP - d)))
    src = jnp.concatenate(
        [aa_embed, clause_pad, jnp.zeros((_ZROWS, _DP), jnp.float32)], axis=0)

    inner = _sc_gather(src, gidx)

    pos = jnp.arange(l_max, dtype=jnp.int32)[None, :]
    mask_indi = (~((pos >= aa_len[:, None]) | (pos >= L))).astype(jnp.float32)

    out = _dense(inner, mask_indi, Wq, bq, Wk, bk, Wg, bg)

    route_idx = map_AS * l_max + jnp.minimum(map_AS_idx, L - 1)
    need = aa_len[map_AS] > 1
    j = jnp.arange(n_as, dtype=jnp.int32)
    final_idx = jnp.where(need, route_idx, b_max * l_max + j)
    table = jnp.concatenate([out.reshape(b_max * l_max, _DP), clause_pad],
                            axis=0)
    res = _sc_gather(table, final_idx)
    return res[:, :d]


# speedup vs baseline: 13.0091x; 1.0982x over previous
import functools

import jax
import jax.numpy as jnp
from jax import lax
from jax.experimental import pallas as pl
from jax.experimental.pallas import tpu as pltpu
from jax.experimental.pallas import tpu_sc as plsc

_HEADS = 6
_TOPK = 5
_DP = 384
_G1 = 64
_BB = 128
_NC = 2
_NS = 16
_CHUNK = 128
_ZROWS = 2048


def _pool_body(g, d, *refs):
    m_ref = refs[1 + g]
    o_ref = refs[2 + g]
    pad = jnp.zeros((1, _DP - d), jnp.float32)
    for j in range(g):
        e = refs[1 + j][0]
        m = m_ref[j]
        s = lax.dot_general(m, e, (((1,), (0,)), ((), ())),
                            preferred_element_type=jnp.float32)
        o_ref[j] = jnp.concatenate([s, pad], axis=1)


def _pooling(input_embed, batch_aa_mask, map_AA):
    n_aa, t_dim = batch_aa_mask.shape
    d = input_embed.shape[-1]
    mask3 = batch_aa_mask.reshape(n_aa, 1, t_dim)

    def espec(j):
        return pl.BlockSpec(
            (1, t_dim, d), lambda i, rowa, j=j: (rowa[i * _G1 + j], 0, 0))

    spec = pltpu.PrefetchScalarGridSpec(
        num_scalar_prefetch=1,
        grid=(n_aa // _G1,),
        in_specs=(
            [espec(j) for j in range(_G1)]
            + [pl.BlockSpec((_G1, 1, t_dim), lambda i, rowa: (i, 0, 0))]
        ),
        out_specs=pl.BlockSpec((_G1, 1, _DP), lambda i, rowa: (i, 0, 0)),
    )
    out = pl.pallas_call(
        functools.partial(_pool_body, _G1, d),
        grid_spec=spec,
        out_shape=jax.ShapeDtypeStruct((n_aa, 1, _DP), jnp.float32),
    )(map_AA, *([input_embed] * _G1), mask3)
    return out.reshape(n_aa, _DP)


def _sc_gather(table, idx):
    n = idx.shape[0]
    per_w = n // (_NC * _NS)
    n_chunks = per_w // _CHUNK
    assert per_w % _CHUNK == 0

    @functools.partial(
        pl.kernel,
        mesh=plsc.VectorSubcoreMesh(core_axis_name="c", subcore_axis_name="s"),
        out_type=jax.ShapeDtypeStruct((n, _DP), jnp.float32),
        scratch_types=[
            pltpu.VMEM((_CHUNK,), jnp.int32),
            pltpu.VMEM((_CHUNK, _DP), jnp.float32),
            pltpu.SemaphoreType.DMA,
        ],
    )
    def k(table_hbm, idx_hbm, out_hbm, idx_v, rows_v, sem):
        wid = lax.axis_index("s") * _NC + lax.axis_index("c")
        for c in range(n_chunks):
            base = wid * per_w + c * _CHUNK
            pltpu.sync_copy(idx_hbm.at[pl.ds(base, _CHUNK)], idx_v)
            pltpu.async_copy(table_hbm.at[idx_v], rows_v, sem).wait()
            pltpu.sync_copy(rows_v, out_hbm.at[pl.ds(base, _CHUNK)])

    return k(table, idx)


def _dense_body(l_max, d, x_ref, m_ref, wq_ref, bq_ref, wk_ref, bk_ref,
                wg_ref, bg_ref, o_ref):
    bb = _BB
    x = x_ref[...][:, :, :d]
    xf = x.reshape(bb * l_max, d)
    mask = m_ref[...].reshape(bb, 1, l_max)

    def proj(w_ref, b_ref):
        r = lax.dot_general(xf, w_ref[...], (((1,), (1,)), ((), ())),
                            preferred_element_type=jnp.float32) + b_ref[...]
        return r.reshape(bb, l_max, d)

    q = proj(wq_ref, bq_ref)
    k = proj(wk_ref, bk_ref)
    dk = d // _HEADS
    scale = 1.0 / (dk ** 0.5)
    neg = jnp.float32(-1e9)

    attn_sum = jnp.zeros((bb, l_max, l_max), jnp.float32)
    for h in range(_HEADS):
        qh = q[:, :, h * dk:(h + 1) * dk]
        kh = k[:, :, h * dk:(h + 1) * dk]
        s = lax.dot_general(
            qh, kh, (((2,), (2,)), ((0,), (0,))),
            preferred_element_type=jnp.float32) * scale
        s = jnp.where(mask == 0.0, neg, s)
        s = s - jnp.max(s, axis=2, keepdims=True)
        e = jnp.exp(s)
        attn_sum = attn_sum + e / jnp.sum(e, axis=2, keepdims=True)

    mm = mask.reshape(bb, l_max, 1) * mask
    adj = (attn_sum / _HEADS) * mm

    col = lax.broadcasted_iota(jnp.int32, (bb, l_max, l_max), 2)
    work = adj
    for _ in range(_TOPK - 1):
        mx = jnp.max(work, axis=2, keepdims=True)
        eq = work == mx
        first = jnp.min(jnp.where(eq, col, l_max), axis=2, keepdims=True)
        work = jnp.where(col == first, neg, work)
    kth = jnp.max(work, axis=2, keepdims=True)
    adj = jnp.where(adj >= kth, adj, 0.0)

    denom = jnp.sum(adj, axis=2, keepdims=True) + 1.0
    ax = lax.dot_general(
        adj, x, (((2,), (1,)), ((0,), (0,))),
        preferred_element_type=jnp.float32)
    g = lax.dot_general(
        ax.reshape(bb * l_max, d), wg_ref[...], (((1,), (1,)), ((), ())),
        preferred_element_type=jnp.float32) + bg_ref[...]
    g = jnp.maximum(g.reshape(bb, l_max, d) / denom, 0.0)
    o_ref[...] = jnp.concatenate(
        [g, jnp.zeros((bb, l_max, _DP - d), jnp.float32)], axis=2)


def _dense(inner, mask_indi, Wq, bq, Wk, bk, Wg, bg):
    b_max, l_max = mask_indi.shape
    d = Wq.shape[0]
    mask3 = mask_indi.reshape(b_max, 1, l_max)
    wspec = pl.BlockSpec((d, d), lambda i: (0, 0))
    bspec = pl.BlockSpec((1, d), lambda i: (0, 0))
    out = pl.pallas_call(
        functools.partial(_dense_body, l_max, d),
        grid=(b_max // _BB,),
        in_specs=[
            pl.BlockSpec((_BB, l_max, _DP), lambda i: (i, 0, 0)),
            pl.BlockSpec((_BB, 1, l_max), lambda i: (i, 0, 0)),
            wspec, bspec, wspec, bspec, wspec, bspec,
        ],
        out_specs=pl.BlockSpec((_BB, l_max, _DP), lambda i: (i, 0, 0)),
        out_shape=jax.ShapeDtypeStruct((b_max, l_max, _DP), jnp.float32),
    )(inner.reshape(b_max, l_max, _DP), mask3,
      Wq, bq.reshape(1, d), Wk, bk.reshape(1, d), Wg, bg.reshape(1, d))
    return out


def kernel(input_embed, clause_output, batch_aa_mask, aa_graph_length,
           map_AA, map_AA_idx, map_AS, map_AS_idx,
           Wq, bq, Wk, bk, Wg, bg):
    b_max, t_dim, d = input_embed.shape
    n_aa = map_AA.shape[0]
    n_as = map_AS.shape[0]
    l_max = 24

    map_AA = map_AA.astype(jnp.int32)
    map_AA_idx = map_AA_idx.astype(jnp.int32)
    map_AS = map_AS.astype(jnp.int32)
    map_AS_idx = map_AS_idx.astype(jnp.int32)
    aa_len = aa_graph_length.astype(jnp.int32)

    B = jnp.max(map_AS) + 1
    L = jnp.max(map_AA_idx) + 1

    row_aa = jnp.where(map_AA < B, map_AA, b_max)
    slot1 = row_aa * (l_max + 1) + map_AA_idx
    col_as = jnp.where(map_AS_idx < L, map_AS_idx, l_max)
    slot2 = map_AS * (l_max + 1) + col_as
    pri = jnp.concatenate([jnp.arange(n_aa, dtype=jnp.int32),
                           n_aa + jnp.arange(n_as, dtype=jnp.int32)])
    win = jnp.full(((b_max + 1) * (l_max + 1),), -1, jnp.int32)
    win = win.at[jnp.concatenate([slot1, slot2])].max(pri)
    nslots = (b_max + 1) * (l_max + 1)
    zspread = n_aa + n_as + (jnp.arange(nslots, dtype=jnp.int32) % _ZROWS)
    src_idx = jnp.where(win >= 0, win, zspread)
    gidx = src_idx.reshape(b_max + 1, l_max + 1)[:b_max, :l_max].reshape(-1)

    aa_embed = _pooling(input_embed, batch_aa_mask, map_AA)

    clause_pad = jnp.pad(clause_output, ((0, 0), (0, _DP - d)))
    src = jnp.concatenate(
        [aa_embed, clause_pad, jnp.zeros((_ZROWS, _DP), jnp.float32)], axis=0)

    inner = _sc_gather(src, gidx)

    pos = jnp.arange(l_max, dtype=jnp.int32)[None, :]
    mask_indi = (~((pos >= aa_len[:, None]) | (pos >= L))).astype(jnp.float32)

    out = _dense(inner, mask_indi, Wq, bq, Wk, bk, Wg, bg)

    route_idx = map_AS * l_max + jnp.minimum(map_AS_idx, L - 1)
    need = aa_len[map_AS] > 1
    j = jnp.arange(n_as, dtype=jnp.int32)
    final_idx = jnp.where(need, route_idx, b_max * l_max + j)
    table = jnp.concatenate([out.reshape(b_max * l_max, _DP), clause_pad],
                            axis=0)
    res = _sc_gather(table, final_idx)
    return res[:, :d]
